# Initial kernel scaffold; baseline (speedup 1.0000x reference)
import jax
import jax.numpy as jnp
from jax.experimental import pallas as pl


def kernel(pooled_Maps, indices, Rectified_FM):
    raise NotImplementedError("write your pallas kernel here")


# speedup vs baseline: 2.9939x; 2.9939x over previous
import functools

import jax
import jax.numpy as jnp
import numpy as np
from jax import lax
from jax.experimental import pallas as pl
from jax.experimental.pallas import tpu as pltpu
from jax.experimental.pallas import tpu_sc as plsc

_B, _H, _W, _C = 4, 112, 112, 96
_HO, _WO = 224, 224
_N = _H * _W * _C
_F = _HO * _WO * _C
_E = _B * _N
_NW = 32
_PER_W = _E // _NW
_K = 10752
_NCHUNK = _PER_W // _K

_RECIP224 = np.float32((1.0 + 2.0**-21) / 224.0)
_RECIP96 = np.float32(1.0 / 96.0)

_mesh = plsc.VectorSubcoreMesh(core_axis_name="c", subcore_axis_name="s")


@functools.partial(
    pl.kernel,
    out_type=(),
    mesh=_mesh,
    scratch_types=[
        pltpu.VMEM((_K,), jnp.int32),
        pltpu.VMEM((_K,), jnp.int32),
        pltpu.VMEM((_K,), jnp.int32),
        pltpu.VMEM((_K,), jnp.float32),
        pltpu.SemaphoreType.DMA,
        pltpu.SemaphoreType.DMA,
    ],
)
def _unpool_scatter(pooled_hbm, idx_hbm, out_ref, idx_v, src_v, dst_v, val_v,
                    sem_g, sem_s):
    wid = lax.axis_index("s") * 2 + lax.axis_index("c")
    b = wid >> 3
    base = wid * _PER_W
    src_off = b * _N
    dst_off = b * _F

    def chunk_body(k, carry):
        cbase = base + k * _K
        pltpu.sync_copy(idx_hbm.at[pl.ds(cbase, _K)], idx_v)

        def vec_body(j, carry2):
            sl = pl.ds(pl.multiple_of(j * 16, 16), 16)
            i = idx_v[sl]
            fi = i.astype(jnp.float32)
            q = (fi * _RECIP96).astype(jnp.int32)
            c = i - q * 96
            qf = q.astype(jnp.float32)
            ho = (qf * _RECIP224).astype(jnp.int32)
            wo = q - ho * 224
            src = ((ho >> 1) * (_W * _C) + (wo >> 1) * _C + c) + src_off
            src_v[sl] = src
            dst_v[sl] = i + dst_off
            return carry2

        lax.fori_loop(0, _K // 16, vec_body, 0, unroll=4)
        pltpu.async_copy(pooled_hbm.at[src_v], val_v, sem_g).wait()
        pltpu.async_copy(val_v, out_ref.at[dst_v], sem_s).wait()
        return carry

    lax.fori_loop(0, _NCHUNK, chunk_body, 0)


def kernel(pooled_Maps, indices, Rectified_FM):
    del Rectified_FM
    pooled_flat = pooled_Maps.reshape(-1)
    idx_flat = indices.reshape(-1)
    out_ref = jax.new_ref(jnp.zeros((_B * _F,), jnp.float32))
    _unpool_scatter(pooled_flat, idx_flat, out_ref)
    return out_ref[...].reshape(_B, _HO, _WO, _C)
